# baseline (speedup 1.0000x reference)
import functools

import jax
import jax.numpy as jnp
from jax import lax
from jax.experimental import pallas as pl
from jax.experimental.pallas import tpu as pltpu
from jax.experimental.pallas import tpu_sc as plsc

EMB_DIM = 256
BATCH = 16384
NUM_CORES = 2
NUM_SUBCORES = 16
NUM_WORKERS = NUM_CORES * NUM_SUBCORES
ROWS_PER_WORKER = BATCH // NUM_WORKERS
CHUNK = 64
N_CHUNKS = ROWS_PER_WORKER // CHUNK
NBUF_T = 3
NBUF_S = 4


def _gather_body(ids_hbm, table_hbm, out_hbm, idx_v, sbuf, *rest):
    tbufs = rest[:NBUF_T]
    gsems = rest[NBUF_T:2 * NBUF_T]
    csems = rest[2 * NBUF_T:2 * NBUF_T + NBUF_S]
    osems = rest[2 * NBUF_T + NBUF_S:2 * NBUF_T + 2 * NBUF_S]
    sid = lax.axis_index("s")
    wid = sid * NUM_CORES + lax.axis_index("c")
    base = wid * ROWS_PER_WORKER
    pltpu.sync_copy(ids_hbm.at[pl.ds(base, ROWS_PER_WORKER)], idx_v)

    gath = [None] * N_CHUNKS
    cross = [None] * N_CHUNKS
    outd = [None] * N_CHUNKS

    def do_gather(c):
        bt = c % NBUF_T
        gath[c] = pltpu.async_copy(
            table_hbm.at[idx_v.at[pl.ds(c * CHUNK, CHUNK)]],
            tbufs[bt], gsems[bt])

    def do_cross(c):
        bs = c % NBUF_S
        if c - NBUF_S >= 0:
            outd[c - NBUF_S].wait()
        gath[c].wait()
        cross[c] = pltpu.async_copy(
            tbufs[c % NBUF_T], sbuf.at[sid, bs], csems[bs])

    def do_out(c):
        bs = c % NBUF_S
        cross[c].wait()
        outd[c] = pltpu.async_copy(
            sbuf.at[sid, bs], out_hbm.at[pl.ds(base + c * CHUNK, CHUNK)],
            osems[bs])

    for t in range(N_CHUNKS + 2):
        if t < N_CHUNKS:
            do_gather(t)
        if 0 <= t - 1 < N_CHUNKS:
            do_cross(t - 1)
        if 0 <= t - 2 < N_CHUNKS:
            do_out(t - 2)
    for c in range(max(0, N_CHUNKS - NBUF_S), N_CHUNKS):
        outd[c].wait()


_gather_kernel = functools.partial(
    pl.kernel,
    out_type=jax.ShapeDtypeStruct((BATCH, EMB_DIM), jnp.float32),
    mesh=plsc.VectorSubcoreMesh(core_axis_name="c", subcore_axis_name="s"),
    scratch_types=(
        [pltpu.VMEM((ROWS_PER_WORKER,), jnp.int32),
         pltpu.VMEM_SHARED((NUM_SUBCORES, NBUF_S, CHUNK, EMB_DIM),
                           jnp.float32)]
        + [pltpu.VMEM((CHUNK, EMB_DIM), jnp.float32) for _ in range(NBUF_T)]
        + [pltpu.SemaphoreType.DMA for _ in range(2 * NBUF_T + 2 * NBUF_S)]
    ),
)(_gather_body)


def kernel(ids, token_emb):
    return _gather_kernel(ids.astype(jnp.int32), token_emb)


# speedup vs baseline: 1.0136x; 1.0136x over previous
import functools

import jax
import jax.numpy as jnp
from jax import lax
from jax.experimental import pallas as pl
from jax.experimental.pallas import tpu as pltpu
from jax.experimental.pallas import tpu_sc as plsc

EMB_DIM = 256
BATCH = 16384
NUM_CORES = 2
NUM_SUBCORES = 16
NUM_WORKERS = NUM_CORES * NUM_SUBCORES
ROWS_PER_WORKER = BATCH // NUM_WORKERS
CHUNK = 32
N_CHUNKS = ROWS_PER_WORKER // CHUNK
NBUF = 15
PRIME = 10


def _gather_body(ids_hbm, table_hbm, out_hbm, idx_v, *rest):
    bufs = rest[:NBUF]
    gsems = rest[NBUF:2 * NBUF]
    wsems = rest[2 * NBUF:3 * NBUF]
    wid = lax.axis_index("s") * NUM_CORES + lax.axis_index("c")
    base = wid * ROWS_PER_WORKER
    pltpu.sync_copy(ids_hbm.at[pl.ds(base, ROWS_PER_WORKER)], idx_v)

    def gather(c):
        b = c % NBUF
        return pltpu.async_copy(
            table_hbm.at[idx_v.at[pl.ds(c * CHUNK, CHUNK)]], bufs[b], gsems[b])

    def write(c):
        b = c % NBUF
        return pltpu.async_copy(
            bufs[b], out_hbm.at[pl.ds(base + c * CHUNK, CHUNK)], wsems[b])

    gathers = [None] * N_CHUNKS
    writes = [None] * N_CHUNKS
    for c in range(min(PRIME, N_CHUNKS)):
        gathers[c] = gather(c)
    for c in range(N_CHUNKS):
        g = c + PRIME
        if g < N_CHUNKS:
            if g - NBUF >= 0:
                writes[g - NBUF].wait()
            gathers[g] = gather(g)
        gathers[c].wait()
        writes[c] = write(c)
    for c in range(max(0, N_CHUNKS - NBUF), N_CHUNKS):
        writes[c].wait()


_gather_kernel = functools.partial(
    pl.kernel,
    out_type=jax.ShapeDtypeStruct((BATCH, EMB_DIM), jnp.float32),
    mesh=plsc.VectorSubcoreMesh(core_axis_name="c", subcore_axis_name="s"),
    scratch_types=(
        [pltpu.VMEM((ROWS_PER_WORKER,), jnp.int32)]
        + [pltpu.VMEM((CHUNK, EMB_DIM), jnp.float32) for _ in range(NBUF)]
        + [pltpu.SemaphoreType.DMA for _ in range(2 * NBUF)]
    ),
)(_gather_body)


def kernel(ids, token_emb):
    return _gather_kernel(ids.astype(jnp.int32), token_emb)


# speedup vs baseline: 1.0253x; 1.0116x over previous
import functools

import jax
import jax.numpy as jnp
from jax import lax
from jax.experimental import pallas as pl
from jax.experimental.pallas import tpu as pltpu
from jax.experimental.pallas import tpu_sc as plsc

EMB_DIM = 256
BATCH = 16384
NUM_CORES = 2
NUM_SUBCORES = 16
NUM_WORKERS = NUM_CORES * NUM_SUBCORES
ROWS_PER_WORKER = BATCH // NUM_WORKERS
CHUNK = 32
N_CHUNKS = ROWS_PER_WORKER // CHUNK
NBUF = 15
PRIME = 6


def _gather_body(ids_hbm, table_hbm, out_hbm, idx_v, *rest):
    bufs = rest[:NBUF]
    gsems = rest[NBUF:2 * NBUF]
    wsems = rest[2 * NBUF:3 * NBUF]
    wid = lax.axis_index("c") * NUM_SUBCORES + lax.axis_index("s")
    base = wid * ROWS_PER_WORKER
    pltpu.sync_copy(ids_hbm.at[pl.ds(base, ROWS_PER_WORKER)], idx_v)

    def gather(c):
        b = c % NBUF
        return pltpu.async_copy(
            table_hbm.at[idx_v.at[pl.ds(c * CHUNK, CHUNK)]], bufs[b], gsems[b])

    def write(c):
        b = c % NBUF
        return pltpu.async_copy(
            bufs[b], out_hbm.at[pl.ds(base + c * CHUNK, CHUNK)], wsems[b])

    gathers = [None] * N_CHUNKS
    writes = [None] * N_CHUNKS
    for c in range(min(PRIME, N_CHUNKS)):
        gathers[c] = gather(c)
    for c in range(N_CHUNKS):
        g = c + PRIME
        if g < N_CHUNKS:
            if g - NBUF >= 0:
                writes[g - NBUF].wait()
            gathers[g] = gather(g)
        gathers[c].wait()
        writes[c] = write(c)
    for c in range(max(0, N_CHUNKS - NBUF), N_CHUNKS):
        writes[c].wait()


_gather_kernel = functools.partial(
    pl.kernel,
    out_type=jax.ShapeDtypeStruct((BATCH, EMB_DIM), jnp.float32),
    mesh=plsc.VectorSubcoreMesh(core_axis_name="c", subcore_axis_name="s"),
    scratch_types=(
        [pltpu.VMEM((ROWS_PER_WORKER,), jnp.int32)]
        + [pltpu.VMEM((CHUNK, EMB_DIM), jnp.float32) for _ in range(NBUF)]
        + [pltpu.SemaphoreType.DMA for _ in range(2 * NBUF)]
    ),
)(_gather_body)


def kernel(ids, token_emb):
    return _gather_kernel(ids.astype(jnp.int32), token_emb)


# speedup vs baseline: 1.0445x; 1.0188x over previous
import functools

import jax
import jax.numpy as jnp
from jax import lax
from jax.experimental import pallas as pl
from jax.experimental.pallas import tpu as pltpu
from jax.experimental.pallas import tpu_sc as plsc

EMB_DIM = 256
BATCH = 16384
NUM_CORES = 2
NUM_SUBCORES = 16
NUM_WORKERS = NUM_CORES * NUM_SUBCORES
ROWS_PER_WORKER = BATCH // NUM_WORKERS
CHUNK = 64
N_CHUNKS = ROWS_PER_WORKER // CHUNK
NBUF = 7
PRIME = 4


def _gather_body(ids_hbm, table_hbm, out_hbm, idx_v, *rest):
    bufs = rest[:NBUF]
    gsems = rest[NBUF:2 * NBUF]
    wsems = rest[2 * NBUF:3 * NBUF]
    wid = lax.axis_index("c") * NUM_SUBCORES + lax.axis_index("s")
    base = wid * ROWS_PER_WORKER
    pltpu.sync_copy(ids_hbm.at[pl.ds(base, ROWS_PER_WORKER)], idx_v)

    def gather(c):
        b = c % NBUF
        return pltpu.async_copy(
            table_hbm.at[idx_v.at[pl.ds(c * CHUNK, CHUNK)]], bufs[b], gsems[b])

    def write(c):
        b = c % NBUF
        return pltpu.async_copy(
            bufs[b], out_hbm.at[pl.ds(base + c * CHUNK, CHUNK)], wsems[b])

    gathers = [None] * N_CHUNKS
    writes = [None] * N_CHUNKS
    for c in range(min(PRIME, N_CHUNKS)):
        gathers[c] = gather(c)
    for c in range(N_CHUNKS):
        g = c + PRIME
        if g < N_CHUNKS:
            if g - NBUF >= 0:
                writes[g - NBUF].wait()
            gathers[g] = gather(g)
        gathers[c].wait()
        writes[c] = write(c)
    for c in range(max(0, N_CHUNKS - NBUF), N_CHUNKS):
        writes[c].wait()


_gather_kernel = functools.partial(
    pl.kernel,
    out_type=jax.ShapeDtypeStruct((BATCH, EMB_DIM), jnp.float32),
    mesh=plsc.VectorSubcoreMesh(core_axis_name="c", subcore_axis_name="s"),
    scratch_types=(
        [pltpu.VMEM((ROWS_PER_WORKER,), jnp.int32)]
        + [pltpu.VMEM((CHUNK, EMB_DIM), jnp.float32) for _ in range(NBUF)]
        + [pltpu.SemaphoreType.DMA for _ in range(2 * NBUF)]
    ),
)(_gather_body)


def kernel(ids, token_emb):
    return _gather_kernel(ids.astype(jnp.int32), token_emb)
